# baseline (speedup 1.0000x reference)
import functools

import jax
import jax.numpy as jnp
from jax import lax
from jax.experimental import pallas as pl
from jax.experimental.pallas import tpu as pltpu
from jax.experimental.pallas import tpu_sc as plsc

D_MODEL = 512
B, L = 4096, 200
N = B * L
NC, NS = 2, 16
NW = NC * NS
PER_W = N // NW
CHUNK = 80
N_CHUNKS = PER_W // CHUNK

_mesh = plsc.VectorSubcoreMesh(core_axis_name="c", subcore_axis_name="s")


@functools.partial(
    pl.kernel,
    out_type=jax.ShapeDtypeStruct((N, D_MODEL), jnp.float32),
    mesh=_mesh,
    scratch_types=[
        pltpu.VMEM((PER_W,), jnp.int32),
        pltpu.VMEM((2, CHUNK, D_MODEL), jnp.float32),
        pltpu.SemaphoreType.DMA,
        pltpu.SemaphoreType.DMA,
        pltpu.SemaphoreType.DMA,
        pltpu.SemaphoreType.DMA,
    ],
)
def _embed_sc(idx_hbm, table_hbm, out_hbm, idx_v, rows_v, gsem0, gsem1, osem0, osem1):
    gsems = (gsem0, gsem1)
    osems = (osem0, osem1)
    wid = lax.axis_index("s") * NC + lax.axis_index("c")
    base = wid * PER_W
    pltpu.sync_copy(idx_hbm.at[pl.ds(base, PER_W)], idx_v)

    def g_src(g):
        return table_hbm.at[idx_v.at[pl.ds(g * CHUNK, CHUNK)]]

    def o_dst(g):
        return out_hbm.at[pl.ds(base + g * CHUNK, CHUNK)]

    pltpu.async_copy(g_src(0), rows_v.at[0], gsems[0])

    @pl.loop(0, N_CHUNKS, step=2)
    def _outer(gg):
        for b in range(2):
            g = gg + b
            nb = 1 - b

            @pl.when(g > 0)
            def _():
                pltpu.make_async_copy(rows_v.at[nb], o_dst(g - 1), osems[nb]).wait()

            @pl.when(g + 1 < N_CHUNKS)
            def _():
                pltpu.async_copy(g_src(g + 1), rows_v.at[nb], gsems[nb])

            pltpu.make_async_copy(g_src(g), rows_v.at[b], gsems[b]).wait()
            pltpu.async_copy(rows_v.at[b], o_dst(g), osems[b])

    pltpu.make_async_copy(rows_v.at[1], o_dst(N_CHUNKS - 1), osems[1]).wait()


def kernel(x, holiday_embed):
    idx = x[:, :, -1].reshape(N)
    out = _embed_sc(idx, holiday_embed)
    return out.reshape(B, L, D_MODEL)


# speedup vs baseline: 1.6227x; 1.6227x over previous
import functools

import jax
import jax.numpy as jnp
from jax import lax
from jax.experimental import pallas as pl
from jax.experimental.pallas import tpu as pltpu
from jax.experimental.pallas import tpu_sc as plsc

D_MODEL = 512
TAB_ROWS = 24
B, L = 4096, 200
N = B * L
NC, NS = 2, 16
NW = NC * NS
PER_W = N // NW
CHUNK = 80
N_CHUNKS = PER_W // CHUNK
LANES = 16
D_BLKS = D_MODEL // LANES

_mesh = plsc.VectorSubcoreMesh(core_axis_name="c", subcore_axis_name="s")


@functools.partial(
    pl.kernel,
    out_type=jax.ShapeDtypeStruct((N, D_MODEL), jnp.float32),
    mesh=_mesh,
    scratch_types=[
        pltpu.VMEM((PER_W,), jnp.int32),
        pltpu.VMEM((TAB_ROWS * D_MODEL,), jnp.float32),
        pltpu.VMEM((2, CHUNK, D_MODEL), jnp.float32),
        pltpu.SemaphoreType.DMA,
        pltpu.SemaphoreType.DMA,
    ],
)
def _embed_sc(idx_hbm, table_hbm, out_hbm, idx_v, table_v, stage_v, osem0, osem1):
    osems = (osem0, osem1)
    wid = lax.axis_index("s") * NC + lax.axis_index("c")
    base = wid * PER_W
    pltpu.sync_copy(table_hbm, table_v)
    pltpu.sync_copy(idx_hbm.at[pl.ds(base, PER_W)], idx_v)

    def o_dst(g):
        return out_hbm.at[pl.ds(base + g * CHUNK, CHUNK)]

    @pl.loop(0, N_CHUNKS, step=2)
    def _outer(gg):
        for b in range(2):
            g = gg + b

            @pl.when(g > 1)
            def _():
                pltpu.make_async_copy(stage_v.at[b], o_dst(g - 2), osems[b]).wait()

            @pl.loop(0, CHUNK // LANES)
            def _rowgrp(rr):
                iv = idx_v[pl.ds(g * CHUNK + rr * LANES, LANES)]
                for j in range(LANES):
                    r = rr * LANES + j
                    rbase = iv[j] * D_MODEL
                    for d in range(D_BLKS):
                        stage_v[b, r, pl.ds(d * LANES, LANES)] = table_v[
                            pl.ds(rbase + d * LANES, LANES)
                        ]

            pltpu.async_copy(stage_v.at[b], o_dst(g), osems[b])

    pltpu.make_async_copy(stage_v.at[0], o_dst(N_CHUNKS - 2), osems[0]).wait()
    pltpu.make_async_copy(stage_v.at[1], o_dst(N_CHUNKS - 1), osems[1]).wait()


def kernel(x, holiday_embed):
    idx = x[:, :, -1].reshape(N)
    out = _embed_sc(idx, holiday_embed.reshape(TAB_ROWS * D_MODEL))
    return out.reshape(B, L, D_MODEL)
